# Initial kernel scaffold; baseline (speedup 1.0000x reference)
import jax
import jax.numpy as jnp
from jax.experimental import pallas as pl


def kernel(x, edge_index, W1, b1, W2, b2):
    raise NotImplementedError("write your pallas kernel here")


# speedup vs baseline: 1.7732x; 1.7732x over previous
import functools

import jax
import jax.numpy as jnp
from jax import lax
from jax.experimental import pallas as pl
from jax.experimental.pallas import tpu as pltpu
from jax.experimental.pallas import tpu_sc as plsc

N = 100000
D = 128
M = N - 1


_ROWS = 4000


def _elu(v):
    return jnp.where(v > 0, v, jnp.exp(v) - 1.0)


def _mlp_body(x_ref, w1_ref, b1_ref, w2_ref, b2_ref, o_ref):
    h = jnp.dot(x_ref[...], w1_ref[...], preferred_element_type=jnp.float32)
    h = _elu(h + b1_ref[...])
    h = jnp.dot(h, w2_ref[...], preferred_element_type=jnp.float32)
    o_ref[...] = _elu(h + b2_ref[...])


def _mlp(x, W1, b1, W2, b2):
    grid = (N // _ROWS,)
    return pl.pallas_call(
        _mlp_body,
        grid=grid,
        in_specs=[
            pl.BlockSpec((_ROWS, D), lambda i: (i, 0)),
            pl.BlockSpec((D, D), lambda i: (0, 0)),
            pl.BlockSpec((1, D), lambda i: (0, 0)),
            pl.BlockSpec((D, D), lambda i: (0, 0)),
            pl.BlockSpec((1, D), lambda i: (0, 0)),
        ],
        out_specs=pl.BlockSpec((_ROWS, D), lambda i: (i, 0)),
        out_shape=jax.ShapeDtypeStruct((N, D), jnp.float32),
    )(x, W1, b1.reshape(1, D), W2, b2.reshape(1, D))


_NC, _NS = 2, 16
_NW = _NC * _NS
_SUB = 128
_NSUB = -(-N // (_NW * _SUB))
_CPW = _NSUB * _SUB
_PAD = _NW * _CPW - N

_TBASE = (_NW - 1) * _CPW
_TFULL = (M - _TBASE) // _SUB
_TROW0 = _TBASE + _TFULL * _SUB


def _gmax_body(ms_hbm, idx_hbm, trow_hbm, out_hbm, idx_v, trow_v, gat_v,
               lin_v, sem, sem2):
    wid = lax.axis_index("s") * _NC + lax.axis_index("c")
    base = wid * _CPW
    pltpu.sync_copy(idx_hbm.at[wid], idx_v)

    def _rowmax(r, carry):
        for c in range(D // 16):
            sl = pl.ds(c * 16, 16)
            lin_v[r, sl] = jnp.maximum(lin_v[r, sl], gat_v[r, sl])
        return carry

    def _chunk(j, row0):
        row0 = pl.multiple_of(row0, 8)
        g = pltpu.async_copy(ms_hbm.at[idx_v.at[j]], gat_v, sem)
        pltpu.sync_copy(ms_hbm.at[pl.ds(row0, _SUB)], lin_v)
        g.wait()
        lax.fori_loop(0, _SUB, _rowmax, 0)
        pltpu.sync_copy(lin_v, out_hbm.at[pl.ds(row0, _SUB)])

    def _tail_chunk(j):
        pltpu.sync_copy(trow_hbm, trow_v)
        g = pltpu.async_copy(ms_hbm.at[idx_v.at[j]], gat_v, sem)
        pltpu.async_copy(ms_hbm.at[trow_v.at[0]], lin_v, sem2).wait()
        g.wait()
        lax.fori_loop(0, _SUB, _rowmax, 0)
        pltpu.async_copy(lin_v, out_hbm.at[trow_v.at[0]], sem2).wait()

    for j in range(_NSUB):
        row0 = base + j * _SUB
        if j < _TFULL:
            _chunk(j, row0)
        else:
            @pl.when(wid < _NW - 1)
            def _():
                _chunk(j, row0)

            if j == _TFULL:
                @pl.when(wid == _NW - 1)
                def _():
                    _tail_chunk(j)


def _gmax(ms, idx, trow):
    call = functools.partial(
        pl.kernel,
        out_type=jax.ShapeDtypeStruct((M, D), jnp.float32),
        mesh=plsc.VectorSubcoreMesh(
            core_axis_name="c", subcore_axis_name="s",
            num_cores=_NC, num_subcores=_NS),
        scratch_types=[
            pltpu.VMEM((_NSUB, _SUB), jnp.int32),
            pltpu.VMEM((1, _SUB), jnp.int32),
            pltpu.VMEM((_SUB, D), jnp.float32),
            pltpu.VMEM((_SUB, D), jnp.float32),
            pltpu.SemaphoreType.DMA,
            pltpu.SemaphoreType.DMA,
        ],
    )(_gmax_body)
    return call(ms, idx, trow)


def kernel(x, edge_index, W1, b1, W2, b2):
    ms = _mlp(x, W1, b1, W2, b2)
    col0 = edge_index[:, 0].astype(jnp.int32)
    idx = jnp.concatenate(
        [col0[:M], jnp.broadcast_to(col0[M - 1], (_NW * _CPW - M,))]
    ).reshape(_NW, _NSUB, _SUB)
    trow = jnp.minimum(jnp.arange(_TROW0, _TROW0 + _SUB, dtype=jnp.int32),
                       M - 1).reshape(1, _SUB)
    return _gmax(ms, idx, trow)


# speedup vs baseline: 1.9804x; 1.1168x over previous
import functools

import jax
import jax.numpy as jnp
from jax import lax
from jax.experimental import pallas as pl
from jax.experimental.pallas import tpu as pltpu
from jax.experimental.pallas import tpu_sc as plsc

N = 100000
D = 128
M = N - 1


_ROWS = 4000


def _elu(v):
    return jnp.where(v > 0, v, jnp.exp(v) - 1.0)


def _mlp_body(x_ref, w1_ref, b1_ref, w2_ref, b2_ref, o_ref):
    h = jnp.dot(x_ref[...], w1_ref[...], preferred_element_type=jnp.float32)
    h = _elu(h + b1_ref[...])
    h = jnp.dot(h, w2_ref[...], preferred_element_type=jnp.float32)
    o_ref[...] = _elu(h + b2_ref[...])


def _mlp(x, W1, b1, W2, b2):
    grid = (N // _ROWS,)
    return pl.pallas_call(
        _mlp_body,
        grid=grid,
        in_specs=[
            pl.BlockSpec((_ROWS, D), lambda i: (i, 0)),
            pl.BlockSpec((D, D), lambda i: (0, 0)),
            pl.BlockSpec((1, D), lambda i: (0, 0)),
            pl.BlockSpec((D, D), lambda i: (0, 0)),
            pl.BlockSpec((1, D), lambda i: (0, 0)),
        ],
        out_specs=pl.BlockSpec((_ROWS, D), lambda i: (i, 0)),
        out_shape=jax.ShapeDtypeStruct((N, D), jnp.float32),
    )(x, W1, b1.reshape(1, D), W2, b2.reshape(1, D))


_NC, _NS = 2, 16
_NW = _NC * _NS
_SUB = 128
_NSUB = -(-N // (_NW * _SUB))
_CPW = _NSUB * _SUB
_PAD = _NW * _CPW - N

_TBASE = (_NW - 1) * _CPW
_TFULL = (M - _TBASE) // _SUB
_TROW0 = _TBASE + _TFULL * _SUB


def _gmax_body(ms_hbm, idx_hbm, trow_hbm, out_hbm, idx_v, trow_v,
               gat0, gat1, lin0, lin1,
               sg0, sg1, sl0, sl1, ss0, ss1):
    wid = lax.axis_index("s") * _NC + lax.axis_index("c")
    base = wid * _CPW
    pltpu.sync_copy(idx_hbm.at[wid], idx_v)
    pltpu.sync_copy(trow_hbm, trow_v)
    gat = (gat0, gat1)
    lin = (lin0, lin1)
    sg = (sg0, sg1)
    sl = (sl0, sl1)
    ss = (ss0, ss1)

    def _rowmax(lv, gv):
        def body(r, carry):
            for c in range(D // 16):
                s = pl.ds(c * 16, 16)
                lv[r, s] = jnp.maximum(lv[r, s], gv[r, s])
            return carry
        lax.fori_loop(0, _SUB, body, 0)

    def _pipeline(chunks):
        n = len(chunks)

        def _issue(k):
            j, is_tail = chunks[k]
            p = k & 1
            g = pltpu.async_copy(ms_hbm.at[idx_v.at[j]], gat[p], sg[p])
            if is_tail:
                l = pltpu.async_copy(ms_hbm.at[trow_v.at[0]], lin[p], sl[p])
            else:
                row0 = pl.multiple_of(base + j * _SUB, 8)
                l = pltpu.async_copy(ms_hbm.at[pl.ds(row0, _SUB)],
                                     lin[p], sl[p])
            return g, l

        pend_g = [None, None]
        pend_l = [None, None]
        pend_s = [None, None]
        pend_g[0], pend_l[0] = _issue(0)
        for k in range(n):
            j, is_tail = chunks[k]
            p = k & 1
            if k + 1 < n:
                if pend_s[1 - p] is not None:
                    pend_s[1 - p].wait()
                pend_g[1 - p], pend_l[1 - p] = _issue(k + 1)
            pend_g[p].wait()
            pend_l[p].wait()
            _rowmax(lin[p], gat[p])
            if is_tail:
                pend_s[p] = pltpu.async_copy(lin[p], out_hbm.at[trow_v.at[0]],
                                             ss[p])
            else:
                row0 = pl.multiple_of(base + j * _SUB, 8)
                pend_s[p] = pltpu.async_copy(lin[p],
                                             out_hbm.at[pl.ds(row0, _SUB)],
                                             ss[p])
        for p in range(2):
            if pend_s[p] is not None:
                pend_s[p].wait()

    @pl.when(wid < _NW - 1)
    def _():
        _pipeline([(j, False) for j in range(_NSUB)])

    @pl.when(wid == _NW - 1)
    def _():
        _pipeline([(j, False) for j in range(_TFULL)] + [(_TFULL, True)])


def _gmax(ms, idx, trow):
    call = functools.partial(
        pl.kernel,
        out_type=jax.ShapeDtypeStruct((M, D), jnp.float32),
        mesh=plsc.VectorSubcoreMesh(
            core_axis_name="c", subcore_axis_name="s",
            num_cores=_NC, num_subcores=_NS),
        scratch_types=[
            pltpu.VMEM((_NSUB, _SUB), jnp.int32),
            pltpu.VMEM((1, _SUB), jnp.int32),
            pltpu.VMEM((_SUB, D), jnp.float32),
            pltpu.VMEM((_SUB, D), jnp.float32),
            pltpu.VMEM((_SUB, D), jnp.float32),
            pltpu.VMEM((_SUB, D), jnp.float32),
            pltpu.SemaphoreType.DMA,
            pltpu.SemaphoreType.DMA,
            pltpu.SemaphoreType.DMA,
            pltpu.SemaphoreType.DMA,
            pltpu.SemaphoreType.DMA,
            pltpu.SemaphoreType.DMA,
        ],
    )(_gmax_body)
    return call(ms, idx, trow)


def kernel(x, edge_index, W1, b1, W2, b2):
    ms = _mlp(x, W1, b1, W2, b2)
    col0 = edge_index[:, 0].astype(jnp.int32)
    idx = jnp.concatenate(
        [col0[:M], jnp.broadcast_to(col0[M - 1], (_NW * _CPW - M,))]
    ).reshape(_NW, _NSUB, _SUB)
    trow = jnp.minimum(jnp.arange(_TROW0, _TROW0 + _SUB, dtype=jnp.int32),
                       M - 1).reshape(1, _SUB)
    return _gmax(ms, idx, trow)
